# Initial kernel scaffold; baseline (speedup 1.0000x reference)
import jax
import jax.numpy as jnp
from jax.experimental import pallas as pl


def kernel(x, adj, Wq_w, Wq_b, a_w, a_b, lin_w, lin_b):
    raise NotImplementedError("write your pallas kernel here")


# speedup vs baseline: 3.0506x; 3.0506x over previous
import functools

import jax
import jax.numpy as jnp
from jax import lax
from jax.experimental import pallas as pl
from jax.experimental.pallas import tpu as pltpu
from jax.experimental.pallas import tpu_sc as plsc

_N = 10000
_E = 320000
_D = 128
_CHUNK = 128
_NC = 2
_NS = 16
_NW = _NC * _NS
_RPW = 80
_NROWS = _RPW * _NW
_EPAD = _NROWS * _CHUNK
_NPAD = 10240
_TILE_N = _NPAD // _NS


def _matmul_bias(x, w_t, b_row):
    def body(x_ref, w_ref, b_ref, o_ref):
        o_ref[...] = (
            jnp.dot(x_ref[...], w_ref[...], preferred_element_type=jnp.float32)
            + b_ref[...]
        )

    return pl.pallas_call(
        body,
        out_shape=jax.ShapeDtypeStruct((_N, _D), jnp.float32),
    )(x, w_t, b_row)


def _sc_segment_sum(h, s2d, r2d, zrows):
    mesh = plsc.VectorSubcoreMesh(core_axis_name="c", subcore_axis_name="s")

    @functools.partial(
        pl.kernel,
        mesh=mesh,
        out_type=jax.ShapeDtypeStruct((_NC * _NPAD, _D), jnp.float32),
        scratch_types=[
            pltpu.VMEM((_RPW, _CHUNK), jnp.int32),
            pltpu.VMEM((_RPW, _CHUNK), jnp.int32),
            pltpu.VMEM((_CHUNK, _D), jnp.float32),
            pltpu.VMEM_SHARED((_NPAD, _D), jnp.float32),
            pltpu.SemaphoreType.DMA,
        ],
    )
    def k(h_hbm, s_hbm, r_hbm, z_hbm, out_hbm, s_v, r_v, rows_v, acc, sem):
        c = lax.axis_index("c")
        sid = lax.axis_index("s")
        wid = sid * _NC + c

        pltpu.sync_copy(z_hbm, acc.at[pl.ds(sid * _TILE_N, _TILE_N)])

        base = pl.multiple_of(wid * _RPW, 8)
        pltpu.sync_copy(s_hbm.at[pl.ds(base, _RPW)], s_v)
        pltpu.sync_copy(r_hbm.at[pl.ds(base, _RPW)], r_v)

        plsc.subcore_barrier()

        def body(i, carry):
            pltpu.async_copy(h_hbm.at[s_v.at[i]], rows_v, sem).wait()
            pltpu.sync_copy(rows_v, acc.at[r_v.at[i]], add=True)
            return carry

        lax.fori_loop(0, _RPW, body, 0)

        plsc.subcore_barrier()

        pltpu.sync_copy(
            acc.at[pl.ds(sid * _TILE_N, _TILE_N)],
            out_hbm.at[pl.ds(c * _NPAD + sid * _TILE_N, _TILE_N)],
        )

    return k(h, s2d, r2d, zrows)


def _combine_silu(parts):
    def body(p_ref, o_ref):
        t = p_ref[0, pl.ds(0, _N)] + p_ref[1, pl.ds(0, _N)]
        o_ref[...] = t * (1.0 / (1.0 + jnp.exp(-t)))

    return pl.pallas_call(
        body,
        out_shape=jax.ShapeDtypeStruct((_N, _D), jnp.float32),
    )(parts)


def kernel(x, adj, Wq_w, Wq_b, a_w, a_b, lin_w, lin_b):
    npad = _EPAD - _E
    s_pad = jnp.concatenate([adj[0], jnp.zeros((npad,), jnp.int32)])
    r_pad = jnp.concatenate([adj[1], jnp.full((npad,), _N, jnp.int32)])
    s2d = s_pad.reshape(_NROWS, _CHUNK)
    r2d = r_pad.reshape(_NROWS, _CHUNK)
    h = _matmul_bias(x, lin_w.T, lin_b.reshape(1, _D))
    zrows = jnp.zeros((_TILE_N, _D), jnp.float32)
    parts = _sc_segment_sum(h, s2d, r2d, zrows)
    return _combine_silu(parts.reshape(_NC, _NPAD, _D))


# speedup vs baseline: 3.2959x; 1.0804x over previous
import functools

import jax
import jax.numpy as jnp
from jax import lax
from jax.experimental import pallas as pl
from jax.experimental.pallas import tpu as pltpu
from jax.experimental.pallas import tpu_sc as plsc

_N = 10000
_E = 320000
_D = 128
_CHUNK = 128
_NC = 2
_NS = 16
_NW = _NC * _NS
_RPW = 80
_NROWS = _RPW * _NW
_EPAD = _NROWS * _CHUNK
_NPAD = 10240
_BLK = 16
_TILE_N = _NPAD // _NS


def _matmul_bias(x, w_t, b_row):
    def body(x_ref, w_ref, b_ref, o_ref):
        o_ref[...] = (
            jnp.dot(x_ref[...], w_ref[...], preferred_element_type=jnp.float32)
            + b_ref[...]
        )

    return pl.pallas_call(
        body,
        out_shape=jax.ShapeDtypeStruct((_N, _D), jnp.float32),
    )(x, w_t, b_row)


def _sc_segment_sum(h, s2d, r2d, zrows):
    mesh = plsc.VectorSubcoreMesh(core_axis_name="c", subcore_axis_name="s")

    @functools.partial(
        pl.kernel,
        mesh=mesh,
        out_type=jax.ShapeDtypeStruct((_NC * _NPAD, _D), jnp.float32),
        scratch_types=[
            pltpu.VMEM((_BLK, _CHUNK), jnp.int32),
            pltpu.VMEM((_BLK, _CHUNK), jnp.int32),
            pltpu.VMEM((_CHUNK, _D), jnp.float32),
            pltpu.VMEM((_CHUNK, _D), jnp.float32),
            pltpu.VMEM_SHARED((_NPAD, _D), jnp.float32),
            pltpu.SemaphoreType.DMA,
            pltpu.SemaphoreType.DMA,
        ],
    )
    def k(h_hbm, s_hbm, r_hbm, z_hbm, out_hbm, s_v, r_v, rows0, rows1, acc,
          sem0, sem1):
        c = lax.axis_index("c")
        sid = lax.axis_index("s")
        wid = sid * _NC + c

        pltpu.sync_copy(z_hbm, acc.at[pl.ds(sid * _TILE_N, _TILE_N)])

        plsc.subcore_barrier()

        rows = (rows0, rows1)
        sems = (sem0, sem1)

        def block(g, carry):
            blk = pl.multiple_of(wid * _RPW + g * _BLK, 8)
            pltpu.sync_copy(s_hbm.at[pl.ds(blk, _BLK)], s_v)
            pltpu.sync_copy(r_hbm.at[pl.ds(blk, _BLK)], r_v)

            pltpu.async_copy(h_hbm.at[s_v.at[0]], rows0, sem0)
            pltpu.async_copy(h_hbm.at[s_v.at[1]], rows1, sem1)
            for j in range(_BLK):
                b = j % 2
                pltpu.make_async_copy(h_hbm.at[s_v.at[j]], rows[b],
                                      sems[b]).wait()
                pltpu.sync_copy(rows[b], acc.at[r_v.at[j]], add=True)
                if j + 2 < _BLK:
                    pltpu.async_copy(h_hbm.at[s_v.at[j + 2]], rows[b], sems[b])
            return carry

        lax.fori_loop(0, _RPW // _BLK, block, 0)

        plsc.subcore_barrier()

        pltpu.sync_copy(
            acc.at[pl.ds(sid * _TILE_N, _TILE_N)],
            out_hbm.at[pl.ds(c * _NPAD + sid * _TILE_N, _TILE_N)],
        )

    return k(h, s2d, r2d, zrows)


def _combine_silu(parts):
    def body(p_ref, o_ref):
        t = p_ref[0, pl.ds(0, _N)] + p_ref[1, pl.ds(0, _N)]
        o_ref[...] = t * (1.0 / (1.0 + jnp.exp(-t)))

    return pl.pallas_call(
        body,
        out_shape=jax.ShapeDtypeStruct((_N, _D), jnp.float32),
    )(parts)


def kernel(x, adj, Wq_w, Wq_b, a_w, a_b, lin_w, lin_b):
    npad = _EPAD - _E
    s_pad = jnp.concatenate([adj[0], jnp.zeros((npad,), jnp.int32)])
    r_pad = jnp.concatenate([adj[1], jnp.full((npad,), _N, jnp.int32)])
    s2d = s_pad.reshape(_NROWS, _CHUNK)
    r2d = r_pad.reshape(_NROWS, _CHUNK)
    h = _matmul_bias(x, lin_w.T, lin_b.reshape(1, _D))
    zrows = jnp.zeros((_TILE_N, _D), jnp.float32)
    parts = _sc_segment_sum(h, s2d, r2d, zrows)
    return _combine_silu(parts.reshape(_NC, _NPAD, _D))


# speedup vs baseline: 10.6772x; 3.2395x over previous
import functools

import jax
import jax.numpy as jnp
from jax import lax
from jax.experimental import pallas as pl
from jax.experimental.pallas import tpu as pltpu
from jax.experimental.pallas import tpu_sc as plsc

_N = 10000
_E = 320000
_D = 128
_CHUNK = 128
_NC = 2
_NS = 16
_NW = _NC * _NS
_RPW = 80
_NROWS = _RPW * _NW
_EPAD = _NROWS * _CHUNK
_NPAD = 10240
_BLK = 16
_TILE_N = _NPAD // _NS


def _matmul_bias(x, w_t, b_row):
    def body(x_ref, w_ref, b_ref, o_ref):
        o_ref[...] = (
            jnp.dot(x_ref[...], w_ref[...], preferred_element_type=jnp.float32)
            + b_ref[...]
        )

    return pl.pallas_call(
        body,
        out_shape=jax.ShapeDtypeStruct((_N, _D), jnp.float32),
    )(x, w_t, b_row)


def _sc_segment_sum(h, s2d, r2d, zrows):
    mesh = plsc.VectorSubcoreMesh(core_axis_name="c", subcore_axis_name="s")

    @functools.partial(
        pl.kernel,
        mesh=mesh,
        out_type=jax.ShapeDtypeStruct((_NC * _NPAD, _D), jnp.float32),
        scratch_types=[
            pltpu.VMEM((_BLK, _CHUNK), jnp.int32),
            pltpu.VMEM((_BLK, _CHUNK), jnp.int32),
            pltpu.VMEM((_CHUNK, _D), jnp.float32),
            pltpu.VMEM((_CHUNK, _D), jnp.float32),
            pltpu.VMEM_SHARED((_NPAD, _D), jnp.float32),
            pltpu.SemaphoreType.DMA,
            pltpu.SemaphoreType.DMA,
        ],
    )
    def k(h_hbm, s_hbm, r_hbm, z_hbm, out_hbm, s_v, r_v, rows0, rows1, acc,
          sem0, sem1):
        c = lax.axis_index("c")
        sid = lax.axis_index("s")
        wid = sid * _NC + c

        pltpu.sync_copy(z_hbm, acc.at[pl.ds(sid * _TILE_N, _TILE_N)])

        plsc.subcore_barrier()

        rows = (rows0, rows1)
        sems = (sem0, sem1)

        def block(g, carry):
            blk = pl.multiple_of(wid * _RPW + g * _BLK, 8)
            pltpu.sync_copy(s_hbm.at[pl.ds(blk, _BLK)], s_v)
            pltpu.sync_copy(r_hbm.at[pl.ds(blk, _BLK)], r_v)

            pltpu.async_copy(h_hbm.at[s_v.at[0]], rows0, sem0)
            pltpu.async_copy(h_hbm.at[s_v.at[1]], rows1, sem1)
            for j in range(_BLK):
                b = j % 2
                pltpu.make_async_copy(h_hbm.at[s_v.at[j]], rows[b],
                                      sems[b]).wait()
                pltpu.sync_copy(rows[b], acc.at[r_v.at[j]], add=True)
                if j + 2 < _BLK:
                    pltpu.async_copy(h_hbm.at[s_v.at[j + 2]], rows[b], sems[b])
            return carry

        lax.fori_loop(0, _RPW // _BLK, block, 0)

        plsc.subcore_barrier()

        pltpu.sync_copy(
            acc.at[pl.ds(sid * _TILE_N, _TILE_N)],
            out_hbm.at[pl.ds(c * _NPAD + sid * _TILE_N, _TILE_N)],
        )

    return k(h, s2d, r2d, zrows)


def _combine_silu(parts):
    def body(p_ref, o_ref):
        t = p_ref[0, pl.ds(0, _N)] + p_ref[1, pl.ds(0, _N)]
        o_ref[...] = t * (1.0 / (1.0 + jnp.exp(-t)))

    return pl.pallas_call(
        body,
        out_shape=jax.ShapeDtypeStruct((_N, _D), jnp.float32),
    )(parts)


def kernel(x, adj, Wq_w, Wq_b, a_w, a_b, lin_w, lin_b):
    npad = _EPAD - _E
    pad_i = jnp.arange(npad, dtype=jnp.int32)
    s_pad = jnp.concatenate([adj[0], pad_i % _N])
    r_pad = jnp.concatenate([adj[1], _N + pad_i % (_NPAD - _N)])
    s2d = s_pad.reshape(_NROWS, _CHUNK)
    r2d = r_pad.reshape(_NROWS, _CHUNK)
    h = _matmul_bias(x, lin_w.T, lin_b.reshape(1, _D))
    zrows = jnp.zeros((_TILE_N, _D), jnp.float32)
    parts = _sc_segment_sum(h, s2d, r2d, zrows)
    return _combine_silu(parts.reshape(_NC, _NPAD, _D))


# speedup vs baseline: 11.2385x; 1.0526x over previous
import functools

import jax
import jax.numpy as jnp
from jax import lax
from jax.experimental import pallas as pl
from jax.experimental.pallas import tpu as pltpu
from jax.experimental.pallas import tpu_sc as plsc

_N = 10000
_E = 320000
_D = 128
_CHUNK = 128
_NC = 2
_NS = 16
_NW = _NC * _NS
_RPW = 80
_NROWS = _RPW * _NW
_EPAD = _NROWS * _CHUNK
_NPAD = 10240
_BLK = 40
_TILE_N = _NPAD // _NS


def _matmul_bias(x, w_t, b_row):
    def body(x_ref, w_ref, b_ref, o_ref):
        o_ref[...] = (
            jnp.dot(x_ref[...], w_ref[...], preferred_element_type=jnp.float32)
            + b_ref[...]
        )

    return pl.pallas_call(
        body,
        out_shape=jax.ShapeDtypeStruct((_N, _D), jnp.float32),
    )(x, w_t, b_row)


def _sc_segment_sum(h, s2d, r2d, zrows):
    mesh = plsc.VectorSubcoreMesh(core_axis_name="c", subcore_axis_name="s")

    @functools.partial(
        pl.kernel,
        mesh=mesh,
        out_type=jax.ShapeDtypeStruct((_NC * _NPAD, _D), jnp.float32),
        scratch_types=[
            pltpu.VMEM((_BLK, _CHUNK), jnp.int32),
            pltpu.VMEM((_BLK, _CHUNK), jnp.int32),
            pltpu.VMEM((_CHUNK, _D), jnp.float32),
            pltpu.VMEM((_CHUNK, _D), jnp.float32),
            pltpu.VMEM_SHARED((_NPAD, _D), jnp.float32),
            pltpu.SemaphoreType.DMA,
            pltpu.SemaphoreType.DMA,
        ],
    )
    def k(h_hbm, s_hbm, r_hbm, z_hbm, out_hbm, s_v, r_v, rows0, rows1, acc,
          sem0, sem1):
        c = lax.axis_index("c")
        sid = lax.axis_index("s")
        wid = sid * _NC + c

        pltpu.sync_copy(z_hbm, acc.at[pl.ds(sid * _TILE_N, _TILE_N)])

        plsc.subcore_barrier()

        rows = (rows0, rows1)
        sems = (sem0, sem1)

        def block(g, carry):
            blk = pl.multiple_of(wid * _RPW + g * _BLK, 8)
            pltpu.sync_copy(s_hbm.at[pl.ds(blk, _BLK)], s_v)
            pltpu.sync_copy(r_hbm.at[pl.ds(blk, _BLK)], r_v)

            pltpu.async_copy(h_hbm.at[s_v.at[0]], rows0, sem0)
            pltpu.async_copy(h_hbm.at[s_v.at[1]], rows1, sem1)
            for j in range(_BLK):
                b = j % 2
                pltpu.make_async_copy(h_hbm.at[s_v.at[j]], rows[b],
                                      sems[b]).wait()
                pltpu.sync_copy(rows[b], acc.at[r_v.at[j]], add=True)
                if j + 2 < _BLK:
                    pltpu.async_copy(h_hbm.at[s_v.at[j + 2]], rows[b], sems[b])
            return carry

        lax.fori_loop(0, _RPW // _BLK, block, 0)

        plsc.subcore_barrier()

        pltpu.sync_copy(
            acc.at[pl.ds(sid * _TILE_N, _TILE_N)],
            out_hbm.at[pl.ds(c * _NPAD + sid * _TILE_N, _TILE_N)],
        )

    return k(h, s2d, r2d, zrows)


def _combine_silu(parts):
    def body(p_ref, o_ref):
        t = p_ref[0, pl.ds(0, _N)] + p_ref[1, pl.ds(0, _N)]
        o_ref[...] = t * (1.0 / (1.0 + jnp.exp(-t)))

    return pl.pallas_call(
        body,
        out_shape=jax.ShapeDtypeStruct((_N, _D), jnp.float32),
    )(parts)


def kernel(x, adj, Wq_w, Wq_b, a_w, a_b, lin_w, lin_b):
    npad = _EPAD - _E
    pad_i = jnp.arange(npad, dtype=jnp.int32)
    s_pad = jnp.concatenate([adj[0], pad_i % _N])
    r_pad = jnp.concatenate([adj[1], _N + pad_i % (_NPAD - _N)])
    s2d = s_pad.reshape(_NROWS, _CHUNK)
    r2d = r_pad.reshape(_NROWS, _CHUNK)
    h = _matmul_bias(x, lin_w.T, lin_b.reshape(1, _D))
    zrows = jnp.zeros((_TILE_N, _D), jnp.float32)
    parts = _sc_segment_sum(h, s2d, r2d, zrows)
    return _combine_silu(parts.reshape(_NC, _NPAD, _D))
